# Initial kernel scaffold; baseline (speedup 1.0000x reference)
import jax
import jax.numpy as jnp
from jax.experimental import pallas as pl


def kernel(x, W, b):
    raise NotImplementedError("write your pallas kernel here")


# speedup vs baseline: 1.3057x; 1.3057x over previous
import functools

import jax
import jax.numpy as jnp
from jax.experimental import pallas as pl
from jax.experimental.pallas import tpu as pltpu

NUM_EXPERTS = 64
TOP_K = 8
CAPACITY_FACTOR = 1.25
ALPHA = 0.01

BLOCK = 512


def _gate_kernel(x_ref, w_ref, b_ref, idx_ref, wt_ref, aux_ref, load_acc,
                 *, n_tokens):
    i = pl.program_id(0)
    n = pl.num_programs(0)

    x = x_ref[...]
    w = w_ref[...]
    logits = jax.lax.dot_general(
        x, w, (((1,), (1,)), ((), ())),
        preferred_element_type=jnp.float32)
    logits = logits + b_ref[...]

    m = jnp.max(logits, axis=-1, keepdims=True)
    e = jnp.exp(logits - m)
    scores = e / jnp.sum(e, axis=-1, keepdims=True)

    iota = jax.lax.broadcasted_iota(jnp.int32, scores.shape, 1)
    work = scores
    idx_cols = []
    wt_cols = []
    load = jnp.zeros((1, NUM_EXPERTS), jnp.float32)
    for _ in range(TOP_K):
        mx = jnp.max(work, axis=-1, keepdims=True)
        hit = work == mx
        sel = jnp.min(jnp.where(hit, iota, NUM_EXPERTS),
                      axis=-1, keepdims=True)
        idx_cols.append(sel)
        wt_cols.append(mx)
        chosen = iota == sel
        load = load + jnp.sum(chosen.astype(jnp.float32), axis=0,
                              keepdims=True)
        work = jnp.where(chosen, -1.0, work)

    idx = jnp.concatenate(idx_cols, axis=-1)
    wts = jnp.concatenate(wt_cols, axis=-1)
    denom = jnp.sum(wts, axis=-1, keepdims=True) + 1e-9
    wts = wts / denom

    idx_ref[...] = idx
    wt_ref[...] = wts

    @pl.when(i == 0)
    def _init():
        load_acc[...] = jnp.zeros_like(load_acc)

    load_acc[...] += load

    @pl.when(i == n - 1)
    def _finish():
        total = load_acc[...]
        capacity = CAPACITY_FACTOR * (n_tokens * TOP_K) / NUM_EXPERTS
        penalty = jnp.sum(jnp.maximum(total - capacity, 0.0),
                          axis=-1, keepdims=True)
        aux_ref[...] = ALPHA * penalty / NUM_EXPERTS / n_tokens


def kernel(x, W, b):
    batch, seq, d = x.shape
    n_tokens = batch * seq
    xf = x.reshape(n_tokens, d)
    b2 = b.reshape(1, NUM_EXPERTS)
    grid = n_tokens // BLOCK

    idx, wts, aux = pl.pallas_call(
        functools.partial(_gate_kernel, n_tokens=n_tokens),
        grid=(grid,),
        in_specs=[
            pl.BlockSpec((BLOCK, d), lambda i: (i, 0)),
            pl.BlockSpec((NUM_EXPERTS, d), lambda i: (0, 0)),
            pl.BlockSpec((1, NUM_EXPERTS), lambda i: (0, 0)),
        ],
        out_specs=[
            pl.BlockSpec((BLOCK, TOP_K), lambda i: (i, 0)),
            pl.BlockSpec((BLOCK, TOP_K), lambda i: (i, 0)),
            pl.BlockSpec((1, 1), lambda i: (0, 0)),
        ],
        out_shape=[
            jax.ShapeDtypeStruct((n_tokens, TOP_K), jnp.int32),
            jax.ShapeDtypeStruct((n_tokens, TOP_K), jnp.float32),
            jax.ShapeDtypeStruct((1, 1), jnp.float32),
        ],
        scratch_shapes=[pltpu.VMEM((1, NUM_EXPERTS), jnp.float32)],
        compiler_params=pltpu.CompilerParams(
            dimension_semantics=("arbitrary",)),
    )(xf, W, b2)

    return (idx.reshape(batch, seq, TOP_K),
            wts.reshape(batch, seq, TOP_K),
            aux[0, 0])


# speedup vs baseline: 1.9006x; 1.4556x over previous
import functools

import jax
import jax.numpy as jnp
from jax.experimental import pallas as pl
from jax.experimental.pallas import tpu as pltpu

NUM_EXPERTS = 64
TOP_K = 8
CAPACITY_FACTOR = 1.25
ALPHA = 0.01

BLOCK = 512


def _gate_kernel(x_ref, w_ref, b_ref, idx_ref, wt_ref, aux_ref, load_acc,
                 *, n_tokens):
    i = pl.program_id(0)
    n = pl.num_programs(0)

    x = x_ref[...]
    w = w_ref[...]
    logits = jax.lax.dot_general(
        w, x, (((1,), (1,)), ((), ())),
        preferred_element_type=jnp.float32)
    logits = logits + b_ref[...]

    m = jnp.max(logits, axis=0, keepdims=True)
    nv = m - logits
    iota = jax.lax.broadcasted_iota(jnp.int32, nv.shape, 0)

    idx_rows = []
    val_rows = []
    for _ in range(TOP_K):
        mn = jnp.min(nv, axis=0, keepdims=True)
        hit = nv == mn
        sel = jnp.min(jnp.where(hit, iota, NUM_EXPERTS),
                      axis=0, keepdims=True)
        idx_rows.append(sel)
        val_rows.append(mn)
        nv = jnp.where(iota == sel, jnp.inf, nv)

    idx_t = jnp.concatenate(idx_rows, axis=0)
    n8 = jnp.concatenate(val_rows, axis=0)
    wts = jnp.exp(-n8)
    wts = wts / (jnp.sum(wts, axis=0, keepdims=True) + 1e-9)

    idx_ref[...] = idx_t
    wt_ref[...] = wts

    chosen = (nv == jnp.inf).astype(jnp.float32)

    @pl.when(i == 0)
    def _init():
        load_acc[...] = jnp.zeros_like(load_acc)

    load_acc[...] += chosen

    @pl.when(i == n - 1)
    def _finish():
        total = jnp.sum(load_acc[...], axis=1, keepdims=True)
        capacity = CAPACITY_FACTOR * (n_tokens * TOP_K) / NUM_EXPERTS
        penalty = jnp.sum(jnp.maximum(total - capacity, 0.0),
                          axis=0, keepdims=True)
        aux_ref[...] = ALPHA * penalty / NUM_EXPERTS / n_tokens


def kernel(x, W, b):
    batch, seq, d = x.shape
    n_tokens = batch * seq
    xf = x.reshape(n_tokens, d)
    b2 = b.reshape(NUM_EXPERTS, 1)
    grid = n_tokens // BLOCK

    idx_t, wts_t, aux = pl.pallas_call(
        functools.partial(_gate_kernel, n_tokens=n_tokens),
        grid=(grid,),
        in_specs=[
            pl.BlockSpec((BLOCK, d), lambda i: (i, 0)),
            pl.BlockSpec((NUM_EXPERTS, d), lambda i: (0, 0)),
            pl.BlockSpec((NUM_EXPERTS, 1), lambda i: (0, 0)),
        ],
        out_specs=[
            pl.BlockSpec((TOP_K, BLOCK), lambda i: (0, i)),
            pl.BlockSpec((TOP_K, BLOCK), lambda i: (0, i)),
            pl.BlockSpec((1, 1), lambda i: (0, 0)),
        ],
        out_shape=[
            jax.ShapeDtypeStruct((TOP_K, n_tokens), jnp.int32),
            jax.ShapeDtypeStruct((TOP_K, n_tokens), jnp.float32),
            jax.ShapeDtypeStruct((1, 1), jnp.float32),
        ],
        scratch_shapes=[pltpu.VMEM((NUM_EXPERTS, BLOCK), jnp.float32)],
        compiler_params=pltpu.CompilerParams(
            dimension_semantics=("arbitrary",)),
    )(xf, W, b2)

    return (idx_t.T.reshape(batch, seq, TOP_K),
            wts_t.T.reshape(batch, seq, TOP_K),
            aux[0, 0])


# speedup vs baseline: 2.0742x; 1.0913x over previous
import functools

import jax
import jax.numpy as jnp
from jax.experimental import pallas as pl
from jax.experimental.pallas import tpu as pltpu

NUM_EXPERTS = 64
TOP_K = 8
CAPACITY_FACTOR = 1.25
ALPHA = 0.01

BLOCK = 1024


def _gate_kernel(x_ref, w_ref, b_ref, idx_ref, wt_ref, aux_ref, load_acc,
                 *, n_tokens):
    i = pl.program_id(0)
    n = pl.num_programs(0)

    x = x_ref[...]
    w = w_ref[...]
    logits = jax.lax.dot_general(
        w, x, (((1,), (1,)), ((), ())),
        preferred_element_type=jnp.float32)
    logits = logits + b_ref[...]

    m = jnp.max(logits, axis=0, keepdims=True)
    nv = m - logits
    iota = jax.lax.broadcasted_iota(jnp.int32, nv.shape, 0)

    idx_rows = []
    val_rows = []
    for _ in range(TOP_K):
        mn = jnp.min(nv, axis=0, keepdims=True)
        hit = nv == mn
        sel = jnp.min(jnp.where(hit, iota, NUM_EXPERTS),
                      axis=0, keepdims=True)
        idx_rows.append(sel)
        val_rows.append(mn)
        nv = jnp.where(iota == sel, jnp.inf, nv)

    idx_t = jnp.concatenate(idx_rows, axis=0)
    n8 = jnp.concatenate(val_rows, axis=0)
    wts = jnp.exp(-n8)
    wts = wts / (jnp.sum(wts, axis=0, keepdims=True) + 1e-9)

    idx_ref[...] = idx_t
    wt_ref[...] = wts

    chosen = (nv == jnp.inf).astype(jnp.float32)

    @pl.when(i == 0)
    def _init():
        load_acc[...] = jnp.zeros_like(load_acc)

    load_acc[...] += chosen

    @pl.when(i == n - 1)
    def _finish():
        total = jnp.sum(load_acc[...], axis=1, keepdims=True)
        capacity = CAPACITY_FACTOR * (n_tokens * TOP_K) / NUM_EXPERTS
        penalty = jnp.sum(jnp.maximum(total - capacity, 0.0),
                          axis=0, keepdims=True)
        aux_ref[...] = ALPHA * penalty / NUM_EXPERTS / n_tokens


def kernel(x, W, b):
    batch, seq, d = x.shape
    n_tokens = batch * seq
    xf = x.reshape(n_tokens, d)
    b2 = b.reshape(NUM_EXPERTS, 1)
    grid = n_tokens // BLOCK

    idx_t, wts_t, aux = pl.pallas_call(
        functools.partial(_gate_kernel, n_tokens=n_tokens),
        grid=(grid,),
        in_specs=[
            pl.BlockSpec((BLOCK, d), lambda i: (i, 0)),
            pl.BlockSpec((NUM_EXPERTS, d), lambda i: (0, 0)),
            pl.BlockSpec((NUM_EXPERTS, 1), lambda i: (0, 0)),
        ],
        out_specs=[
            pl.BlockSpec((TOP_K, BLOCK), lambda i: (0, i)),
            pl.BlockSpec((TOP_K, BLOCK), lambda i: (0, i)),
            pl.BlockSpec((1, 1), lambda i: (0, 0)),
        ],
        out_shape=[
            jax.ShapeDtypeStruct((TOP_K, n_tokens), jnp.int32),
            jax.ShapeDtypeStruct((TOP_K, n_tokens), jnp.float32),
            jax.ShapeDtypeStruct((1, 1), jnp.float32),
        ],
        scratch_shapes=[pltpu.VMEM((NUM_EXPERTS, BLOCK), jnp.float32)],
        compiler_params=pltpu.CompilerParams(
            dimension_semantics=("arbitrary",)),
    )(xf, W, b2)

    return (idx_t.T.reshape(batch, seq, TOP_K),
            wts_t.T.reshape(batch, seq, TOP_K),
            aux[0, 0])
